# Initial kernel scaffold; baseline (speedup 1.0000x reference)
import jax
import jax.numpy as jnp
from jax.experimental import pallas as pl


def kernel(x_temporal, gcn_W0, gcn_b0, gcn_W1, gcn_b1, gcn_W2, gcn_b2, lstm_fw_Wih0, lstm_fw_Whh0, lstm_fw_b0, lstm_bw_Wih0, lstm_bw_Whh0, lstm_bw_b0, lstm_fw_Wih1, lstm_fw_Whh1, lstm_fw_b1, lstm_bw_Wih1, lstm_bw_Whh1, lstm_bw_b1, cls_W1, cls_b1, cls_W2, cls_b2):
    raise NotImplementedError("write your pallas kernel here")


# speedup vs baseline: 23.0458x; 23.0458x over previous
import numpy as np
import jax
import jax.numpy as jnp
from jax.experimental import pallas as pl
from jax.experimental.pallas import tpu as pltpu

_B, _T, _N, _F = 64, 50, 68, 128
_H = 256
_K = 5
_NP = _N + 2 * _K
_G = _B * _T
_GC = 64
_NCLS = 500
_ND = 2 * _K + 1


def _band_coeffs_np():
    deg = np.array([min(_N - 1, i + _K) - max(0, i - _K) + 1 for i in range(_N)],
                   np.float32)
    dis = (1.0 / np.sqrt(deg)).astype(np.float32)
    band = np.zeros((_ND, _N), np.float32)
    for d in range(_ND):
        for i in range(_N):
            j = i - _K + d
            if 0 <= j < _N:
                band[d, i] = dis[i] * dis[j]
    return band


def _gcn_body(x_ref, band_ref, w0, b0, w1, b1, w2, b2, wih, bih, out_ref, hpad):
    zpad = jnp.zeros((_K, _GC, _F), jnp.float32)
    hpad[0:_K] = zpad
    hpad[_K + _N:_NP] = zpad
    h = x_ref[...]
    for w_ref, b_ref in ((w0, b0), (w1, b1), (w2, b2)):
        hw = jnp.reshape(jnp.reshape(h, (_N * _GC, _F)) @ w_ref[...],
                         (_N, _GC, _F))
        hpad[_K:_K + _N] = hw
        acc = jnp.zeros((_N, _GC, _F), jnp.float32)
        for d in range(_ND):
            acc = acc + band_ref[d][:, None, :] * hpad[d:d + _N]
        h = jnp.maximum(acc + b_ref[...][None], 0.0)
    emb = jnp.sum(h, axis=0) * (1.0 / _N)
    out_ref[...] = emb @ wih[...] + bih[...]


def _gcn(xt, band, w0, b0, w1, b1, w2, b2, wih, bih):
    const2 = lambda i: (0, 0)
    const3 = lambda i: (0, 0, 0)
    return pl.pallas_call(
        _gcn_body,
        grid=(_G // _GC,),
        in_specs=[
            pl.BlockSpec((_N, _GC, _F), lambda i: (0, i, 0)),
            pl.BlockSpec((_ND, _N, _F), const3),
            pl.BlockSpec((_F, _F), const2), pl.BlockSpec((1, _F), const2),
            pl.BlockSpec((_F, _F), const2), pl.BlockSpec((1, _F), const2),
            pl.BlockSpec((_F, _F), const2), pl.BlockSpec((1, _F), const2),
            pl.BlockSpec((_F, 8 * _H), const2),
            pl.BlockSpec((1, 8 * _H), const2),
        ],
        out_specs=pl.BlockSpec((_GC, 8 * _H), lambda i: (i, 0)),
        out_shape=jax.ShapeDtypeStruct((_G, 8 * _H), jnp.float32),
        scratch_shapes=[pltpu.VMEM((_NP, _GC, _F), jnp.float32)],
    )(xt, band, w0, b0, w1, b1, w2, b2, wih, bih)


def _lstm_step(g, h_ref, c_ref, whh_ref):
    g = g + h_ref[...] @ whh_ref[...]
    i = jax.nn.sigmoid(g[:, 0:_H])
    f = jax.nn.sigmoid(g[:, _H:2 * _H])
    gg = jnp.tanh(g[:, 2 * _H:3 * _H])
    o = jax.nn.sigmoid(g[:, 3 * _H:4 * _H])
    c = f * c_ref[...] + i * gg
    h = o * jnp.tanh(c)
    c_ref[...] = c
    h_ref[...] = h
    return h


def _rec0_body(gf_ref, gb_ref, whhf, whhb, outf_ref, outb_ref, hf, cf, hb, cb):
    s = pl.program_id(0)

    @pl.when(s == 0)
    def _init():
        z = jnp.zeros((_B, _H), jnp.float32)
        hf[...] = z
        cf[...] = z
        hb[...] = z
        cb[...] = z

    outf_ref[0] = _lstm_step(gf_ref[0], hf, cf, whhf)
    outb_ref[0] = _lstm_step(gb_ref[0], hb, cb, whhb)


def _rec0(gates, whhf, whhb):
    const2 = lambda s: (0, 0)
    return pl.pallas_call(
        _rec0_body,
        grid=(_T,),
        in_specs=[
            pl.BlockSpec((1, _B, 4 * _H), lambda s: (s, 0, 0)),
            pl.BlockSpec((1, _B, 4 * _H), lambda s: (_T - 1 - s, 0, 1)),
            pl.BlockSpec((_H, 4 * _H), const2),
            pl.BlockSpec((_H, 4 * _H), const2),
        ],
        out_specs=[
            pl.BlockSpec((1, _B, _H), lambda s: (s, 0, 0)),
            pl.BlockSpec((1, _B, _H), lambda s: (_T - 1 - s, 0, 0)),
        ],
        out_shape=[
            jax.ShapeDtypeStruct((_T, _B, _H), jnp.float32),
            jax.ShapeDtypeStruct((_T, _B, _H), jnp.float32),
        ],
        scratch_shapes=[pltpu.VMEM((_B, _H), jnp.float32)] * 4,
    )(gates, gates, whhf, whhb)


def _proj_body(xf_ref, xb_ref, wf_ref, wb_ref, b_ref, o_ref):
    o_ref[...] = (xf_ref[...] @ wf_ref[...] + xb_ref[...] @ wb_ref[...]
                  + b_ref[...])


def _proj(xf2d, xb2d, wf, wb, b, rows_per_step):
    n_in, n_out = wf.shape
    return pl.pallas_call(
        _proj_body,
        grid=(xf2d.shape[0] // rows_per_step,),
        in_specs=[
            pl.BlockSpec((rows_per_step, n_in), lambda i: (i, 0)),
            pl.BlockSpec((rows_per_step, n_in), lambda i: (i, 0)),
            pl.BlockSpec((n_in, n_out), lambda i: (0, 0)),
            pl.BlockSpec((n_in, n_out), lambda i: (0, 0)),
            pl.BlockSpec((1, n_out), lambda i: (0, 0)),
        ],
        out_specs=pl.BlockSpec((rows_per_step, n_out), lambda i: (i, 0)),
        out_shape=jax.ShapeDtypeStruct((xf2d.shape[0], n_out), jnp.float32),
    )(xf2d, xb2d, wf, wb, b)


def _rec1_body(gf_ref, gb_ref, whhf, whhb, w1, b1, w2, b2, out_ref,
               hf, cf, hb, cb):
    s = pl.program_id(0)

    @pl.when(s == 0)
    def _init():
        z = jnp.zeros((_B, _H), jnp.float32)
        hf[...] = z
        cf[...] = z
        hb[...] = z
        cb[...] = z

    hfv = _lstm_step(gf_ref[0], hf, cf, whhf)
    hbv = _lstm_step(gb_ref[0], hb, cb, whhb)

    @pl.when(s == _T - 1)
    def _cls():
        hcat = jnp.concatenate([hfv, hbv], axis=1)
        hid = jnp.maximum(hcat @ w1[...] + b1[...], 0.0)
        out_ref[...] = hid @ w2[...] + b2[...]


def _rec1(gates, whhf, whhb, w1, b1, w2, b2):
    const2 = lambda s: (0, 0)
    return pl.pallas_call(
        _rec1_body,
        grid=(_T,),
        in_specs=[
            pl.BlockSpec((1, _B, 4 * _H), lambda s: (s, 0, 0)),
            pl.BlockSpec((1, _B, 4 * _H), lambda s: (_T - 1 - s, 0, 1)),
            pl.BlockSpec((_H, 4 * _H), const2),
            pl.BlockSpec((_H, 4 * _H), const2),
            pl.BlockSpec((2 * _H, _H), const2),
            pl.BlockSpec((1, _H), const2),
            pl.BlockSpec((_H, _NCLS), const2),
            pl.BlockSpec((1, _NCLS), const2),
        ],
        out_specs=pl.BlockSpec((_B, _NCLS), const2),
        out_shape=jax.ShapeDtypeStruct((_B, _NCLS), jnp.float32),
        scratch_shapes=[pltpu.VMEM((_B, _H), jnp.float32)] * 4,
    )(gates, gates, whhf, whhb, w1, b1, w2, b2)


def kernel(x_temporal, gcn_W0, gcn_b0, gcn_W1, gcn_b1, gcn_W2, gcn_b2,
           lstm_fw_Wih0, lstm_fw_Whh0, lstm_fw_b0,
           lstm_bw_Wih0, lstm_bw_Whh0, lstm_bw_b0,
           lstm_fw_Wih1, lstm_fw_Whh1, lstm_fw_b1,
           lstm_bw_Wih1, lstm_bw_Whh1, lstm_bw_b1,
           cls_W1, cls_b1, cls_W2, cls_b2):
    xt = jnp.transpose(x_temporal, (2, 1, 0, 3)).reshape(_N, _G, _F)

    band = jnp.asarray(np.repeat(_band_coeffs_np()[:, :, None], _F, axis=2))

    wih0 = jnp.concatenate([lstm_fw_Wih0.T, lstm_bw_Wih0.T], axis=1)
    bih0 = jnp.concatenate([lstm_fw_b0, lstm_bw_b0])[None]
    gates0 = _gcn(xt, band, gcn_W0, gcn_b0[None], gcn_W1, gcn_b1[None],
                  gcn_W2, gcn_b2[None], wih0, bih0)
    g0 = gates0.reshape(_T, _B, 8 * _H)

    fw0, bw0 = _rec0(g0, lstm_fw_Whh0.T, lstm_bw_Whh0.T)

    wih1 = jnp.concatenate([lstm_fw_Wih1.T, lstm_bw_Wih1.T], axis=1)
    bih1 = jnp.concatenate([lstm_fw_b1, lstm_bw_b1])[None]
    gates1 = _proj(fw0.reshape(_G, _H), bw0.reshape(_G, _H),
                   wih1[:_H], wih1[_H:], bih1, 400)
    g1 = gates1.reshape(_T, _B, 8 * _H)

    return _rec1(g1, lstm_fw_Whh1.T, lstm_bw_Whh1.T,
                 cls_W1, cls_b1[None], cls_W2, cls_b2[None])


# speedup vs baseline: 25.4373x; 1.1038x over previous
import numpy as np
import jax
import jax.numpy as jnp
from jax.experimental import pallas as pl
from jax.experimental.pallas import tpu as pltpu

_B, _T, _N, _F = 64, 50, 68, 128
_H = 256
_K = 5
_NP = _N + 2 * _K
_G = _B * _T
_GC = 64
_NCLS = 500
_ND = 2 * _K + 1


_NPAD = 80


def _deg_isqrt_np():
    deg = np.array([min(_N - 1, i + _K) - max(0, i - _K) + 1 for i in range(_N)],
                   np.float32)
    return (1.0 / np.sqrt(deg)).astype(np.float32)


def _gcn_body(x_ref, disj_ref, disi_ref, w0, b0, w1, b1, w2, b2, wih, bih,
              out_ref):
    h = x_ref[:, 0]
    zlo = jnp.zeros((_B, _K, _F), jnp.float32)
    zhi = jnp.zeros((_B, _NPAD - _N - _K, _F), jnp.float32)
    disj = disj_ref[...][None]
    disi = disi_ref[...][None]
    for w_ref, b_ref in ((w0, b0), (w1, b1), (w2, b2)):
        hp = jnp.concatenate([zlo, h, zhi], axis=1)
        hw = jnp.reshape(jnp.reshape(hp, (_B * _NPAD, _F)) @ w_ref[...],
                         (_B, _NPAD, _F))
        p = hw * disj
        p2 = p[:, 0:_NPAD - 1] + p[:, 1:_NPAD]
        p4 = p2[:, 0:_NPAD - 3] + p2[:, 2:_NPAD - 1]
        p8 = p4[:, 0:_N] + p4[:, 4:_N + 4]
        win = p8 + p2[:, 8:_N + 8] + p[:, 10:_N + 10]
        h = jnp.maximum(win * disi + b_ref[...][None], 0.0)
    emb = jnp.sum(h, axis=1) * (1.0 / _N)
    out_ref[0] = emb @ wih[...] + bih[...]


def _gcn(x4d, disj, disi, w0, b0, w1, b1, w2, b2, wih, bih):
    const2 = lambda s: (0, 0)
    return pl.pallas_call(
        _gcn_body,
        grid=(_T,),
        in_specs=[
            pl.BlockSpec((_B, 1, _N, _F), lambda s: (0, s, 0, 0)),
            pl.BlockSpec((_NPAD, _F), const2),
            pl.BlockSpec((_N, _F), const2),
            pl.BlockSpec((_F, _F), const2), pl.BlockSpec((1, _F), const2),
            pl.BlockSpec((_F, _F), const2), pl.BlockSpec((1, _F), const2),
            pl.BlockSpec((_F, _F), const2), pl.BlockSpec((1, _F), const2),
            pl.BlockSpec((_F, 8 * _H), const2),
            pl.BlockSpec((1, 8 * _H), const2),
        ],
        out_specs=pl.BlockSpec((1, _B, 8 * _H), lambda s: (s, 0, 0)),
        out_shape=jax.ShapeDtypeStruct((_T, _B, 8 * _H), jnp.float32),
    )(x4d, disj, disi, w0, b0, w1, b1, w2, b2, wih, bih)


def _lstm_step(g, h_ref, c_ref, whh_ref):
    g = g + h_ref[...] @ whh_ref[...]
    i = jax.nn.sigmoid(g[:, 0:_H])
    f = jax.nn.sigmoid(g[:, _H:2 * _H])
    gg = jnp.tanh(g[:, 2 * _H:3 * _H])
    o = jax.nn.sigmoid(g[:, 3 * _H:4 * _H])
    c = f * c_ref[...] + i * gg
    h = o * jnp.tanh(c)
    c_ref[...] = c
    h_ref[...] = h
    return h


def _rec0_body(gf_ref, gb_ref, whhf, whhb, outf_ref, outb_ref, hf, cf, hb, cb):
    s = pl.program_id(0)

    @pl.when(s == 0)
    def _init():
        z = jnp.zeros((_B, _H), jnp.float32)
        hf[...] = z
        cf[...] = z
        hb[...] = z
        cb[...] = z

    outf_ref[0] = _lstm_step(gf_ref[0], hf, cf, whhf)
    outb_ref[0] = _lstm_step(gb_ref[0], hb, cb, whhb)


def _rec0(gates, whhf, whhb):
    const2 = lambda s: (0, 0)
    return pl.pallas_call(
        _rec0_body,
        grid=(_T,),
        in_specs=[
            pl.BlockSpec((1, _B, 4 * _H), lambda s: (s, 0, 0)),
            pl.BlockSpec((1, _B, 4 * _H), lambda s: (_T - 1 - s, 0, 1)),
            pl.BlockSpec((_H, 4 * _H), const2),
            pl.BlockSpec((_H, 4 * _H), const2),
        ],
        out_specs=[
            pl.BlockSpec((1, _B, _H), lambda s: (s, 0, 0)),
            pl.BlockSpec((1, _B, _H), lambda s: (_T - 1 - s, 0, 0)),
        ],
        out_shape=[
            jax.ShapeDtypeStruct((_T, _B, _H), jnp.float32),
            jax.ShapeDtypeStruct((_T, _B, _H), jnp.float32),
        ],
        scratch_shapes=[pltpu.VMEM((_B, _H), jnp.float32)] * 4,
    )(gates, gates, whhf, whhb)


def _proj_body(xf_ref, xb_ref, wf_ref, wb_ref, b_ref, o_ref):
    o_ref[...] = (xf_ref[...] @ wf_ref[...] + xb_ref[...] @ wb_ref[...]
                  + b_ref[...])


def _proj(xf2d, xb2d, wf, wb, b, rows_per_step):
    n_in, n_out = wf.shape
    return pl.pallas_call(
        _proj_body,
        grid=(xf2d.shape[0] // rows_per_step,),
        in_specs=[
            pl.BlockSpec((rows_per_step, n_in), lambda i: (i, 0)),
            pl.BlockSpec((rows_per_step, n_in), lambda i: (i, 0)),
            pl.BlockSpec((n_in, n_out), lambda i: (0, 0)),
            pl.BlockSpec((n_in, n_out), lambda i: (0, 0)),
            pl.BlockSpec((1, n_out), lambda i: (0, 0)),
        ],
        out_specs=pl.BlockSpec((rows_per_step, n_out), lambda i: (i, 0)),
        out_shape=jax.ShapeDtypeStruct((xf2d.shape[0], n_out), jnp.float32),
    )(xf2d, xb2d, wf, wb, b)


def _rec1_body(gf_ref, gb_ref, whhf, whhb, w1, b1, w2, b2, out_ref,
               hf, cf, hb, cb):
    s = pl.program_id(0)

    @pl.when(s == 0)
    def _init():
        z = jnp.zeros((_B, _H), jnp.float32)
        hf[...] = z
        cf[...] = z
        hb[...] = z
        cb[...] = z

    hfv = _lstm_step(gf_ref[0], hf, cf, whhf)
    hbv = _lstm_step(gb_ref[0], hb, cb, whhb)

    @pl.when(s == _T - 1)
    def _cls():
        hcat = jnp.concatenate([hfv, hbv], axis=1)
        hid = jnp.maximum(hcat @ w1[...] + b1[...], 0.0)
        out_ref[...] = hid @ w2[...] + b2[...]


def _rec1(gates, whhf, whhb, w1, b1, w2, b2):
    const2 = lambda s: (0, 0)
    return pl.pallas_call(
        _rec1_body,
        grid=(_T,),
        in_specs=[
            pl.BlockSpec((1, _B, 4 * _H), lambda s: (s, 0, 0)),
            pl.BlockSpec((1, _B, 4 * _H), lambda s: (_T - 1 - s, 0, 1)),
            pl.BlockSpec((_H, 4 * _H), const2),
            pl.BlockSpec((_H, 4 * _H), const2),
            pl.BlockSpec((2 * _H, _H), const2),
            pl.BlockSpec((1, _H), const2),
            pl.BlockSpec((_H, _NCLS), const2),
            pl.BlockSpec((1, _NCLS), const2),
        ],
        out_specs=pl.BlockSpec((_B, _NCLS), const2),
        out_shape=jax.ShapeDtypeStruct((_B, _NCLS), jnp.float32),
        scratch_shapes=[pltpu.VMEM((_B, _H), jnp.float32)] * 4,
    )(gates, gates, whhf, whhb, w1, b1, w2, b2)


def kernel(x_temporal, gcn_W0, gcn_b0, gcn_W1, gcn_b1, gcn_W2, gcn_b2,
           lstm_fw_Wih0, lstm_fw_Whh0, lstm_fw_b0,
           lstm_bw_Wih0, lstm_bw_Whh0, lstm_bw_b0,
           lstm_fw_Wih1, lstm_fw_Whh1, lstm_fw_b1,
           lstm_bw_Wih1, lstm_bw_Whh1, lstm_bw_b1,
           cls_W1, cls_b1, cls_W2, cls_b2):
    dis = _deg_isqrt_np()
    disj = np.zeros((_NPAD,), np.float32)
    disj[_K:_K + _N] = dis
    disj = jnp.asarray(np.repeat(disj[:, None], _F, axis=1))
    disi = jnp.asarray(np.repeat(dis[:, None], _F, axis=1))

    wih0 = jnp.concatenate([lstm_fw_Wih0.T, lstm_bw_Wih0.T], axis=1)
    bih0 = jnp.concatenate([lstm_fw_b0, lstm_bw_b0])[None]
    g0 = _gcn(x_temporal, disj, disi, gcn_W0, gcn_b0[None], gcn_W1,
              gcn_b1[None], gcn_W2, gcn_b2[None], wih0, bih0)

    fw0, bw0 = _rec0(g0, lstm_fw_Whh0.T, lstm_bw_Whh0.T)

    wih1 = jnp.concatenate([lstm_fw_Wih1.T, lstm_bw_Wih1.T], axis=1)
    bih1 = jnp.concatenate([lstm_fw_b1, lstm_bw_b1])[None]
    gates1 = _proj(fw0.reshape(_G, _H), bw0.reshape(_G, _H),
                   wih1[:_H], wih1[_H:], bih1, 400)
    g1 = gates1.reshape(_T, _B, 8 * _H)

    return _rec1(g1, lstm_fw_Whh1.T, lstm_bw_Whh1.T,
                 cls_W1, cls_b1[None], cls_W2, cls_b2[None])


# speedup vs baseline: 40.7425x; 1.6017x over previous
import numpy as np
import jax
import jax.numpy as jnp
from jax.experimental import pallas as pl
from jax.experimental.pallas import tpu as pltpu

_B, _T, _N, _F = 64, 50, 68, 128
_H = 256
_K = 5
_NP = _N + 2 * _K
_G = _B * _T
_GC = 64
_NCLS = 500
_ND = 2 * _K + 1


_NP2 = _N + 2 * _K
_TS = 5
_BC = 8
_TR = 10


def _deg_isqrt_np():
    deg = np.array([min(_N - 1, i + _K) - max(0, i - _K) + 1 for i in range(_N)],
                   np.float32)
    return (1.0 / np.sqrt(deg)).astype(np.float32)


def _gcn_body(x_ref, disj_ref, disi_ref, w0, b0, w1, b1, w2, b2, wih, bih,
              out_ref):
    zpad = jnp.zeros((_TS, _K, _BC, _F), jnp.float32)
    disj = disj_ref[...][None, :, None]
    disi = disi_ref[...][None, :, None]
    for tc in range(_T // _TS):
        h = jnp.transpose(x_ref[:, tc * _TS:(tc + 1) * _TS], (1, 2, 0, 3))
        for w_ref, b_ref in ((w0, b0), (w1, b1), (w2, b2)):
            hw = jnp.reshape(jnp.reshape(h, (_TS * _N * _BC, _F)) @ w_ref[...],
                             (_TS, _N, _BC, _F))
            p = jnp.concatenate([zpad, hw * disj, zpad], axis=1)
            p2 = p[:, 0:_NP2 - 1] + p[:, 1:_NP2]
            p4 = p2[:, 0:_NP2 - 3] + p2[:, 2:_NP2 - 1]
            p8 = p4[:, 0:_N] + p4[:, 4:_N + 4]
            win = p8 + p2[:, 8:_N + 8] + p[:, 10:_N + 10]
            h = jnp.maximum(win * disi + b_ref[...][None, None], 0.0)
        emb = jnp.sum(h, axis=1) * (1.0 / _N)
        g = jnp.reshape(emb, (_TS * _BC, _F)) @ wih[...] + bih[...]
        out_ref[tc * _TS:(tc + 1) * _TS] = jnp.reshape(g, (_TS, _BC, 8 * _H))


def _gcn(x4d, disj, disi, w0, b0, w1, b1, w2, b2, wih, bih):
    const2 = lambda s: (0, 0)
    return pl.pallas_call(
        _gcn_body,
        grid=(_B // _BC,),
        in_specs=[
            pl.BlockSpec((_BC, _T, _N, _F), lambda c: (c, 0, 0, 0)),
            pl.BlockSpec((_N, _F), const2),
            pl.BlockSpec((_N, _F), const2),
            pl.BlockSpec((_F, _F), const2), pl.BlockSpec((1, _F), const2),
            pl.BlockSpec((_F, _F), const2), pl.BlockSpec((1, _F), const2),
            pl.BlockSpec((_F, _F), const2), pl.BlockSpec((1, _F), const2),
            pl.BlockSpec((_F, 8 * _H), const2),
            pl.BlockSpec((1, 8 * _H), const2),
        ],
        out_specs=pl.BlockSpec((_T, _BC, 8 * _H), lambda c: (0, c, 0)),
        out_shape=jax.ShapeDtypeStruct((_T, _B, 8 * _H), jnp.float32),
    )(x4d, disj, disi, w0, b0, w1, b1, w2, b2, wih, bih)


def _lstm_step(g, h_ref, c_ref, whh_ref):
    g = g + h_ref[...] @ whh_ref[...]
    i = jax.nn.sigmoid(g[:, 0:_H])
    f = jax.nn.sigmoid(g[:, _H:2 * _H])
    gg = jnp.tanh(g[:, 2 * _H:3 * _H])
    o = jax.nn.sigmoid(g[:, 3 * _H:4 * _H])
    c = f * c_ref[...] + i * gg
    h = o * jnp.tanh(c)
    c_ref[...] = c
    h_ref[...] = h
    return h


def _rec0_body(gf_ref, gb_ref, whhf, whhb, outf_ref, outb_ref, hf, cf, hb, cb):
    s = pl.program_id(0)

    @pl.when(s == 0)
    def _init():
        z = jnp.zeros((_B, _H), jnp.float32)
        hf[...] = z
        cf[...] = z
        hb[...] = z
        cb[...] = z

    for tt in range(_TR):
        outf_ref[tt] = _lstm_step(gf_ref[tt], hf, cf, whhf)
        outb_ref[_TR - 1 - tt] = _lstm_step(gb_ref[_TR - 1 - tt], hb, cb, whhb)


def _rec0(gates, whhf, whhb):
    const2 = lambda s: (0, 0)
    nsteps = _T // _TR
    return pl.pallas_call(
        _rec0_body,
        grid=(nsteps,),
        in_specs=[
            pl.BlockSpec((_TR, _B, 4 * _H), lambda s: (s, 0, 0)),
            pl.BlockSpec((_TR, _B, 4 * _H), lambda s: (nsteps - 1 - s, 0, 1)),
            pl.BlockSpec((_H, 4 * _H), const2),
            pl.BlockSpec((_H, 4 * _H), const2),
        ],
        out_specs=[
            pl.BlockSpec((_TR, _B, _H), lambda s: (s, 0, 0)),
            pl.BlockSpec((_TR, _B, _H), lambda s: (nsteps - 1 - s, 0, 0)),
        ],
        out_shape=[
            jax.ShapeDtypeStruct((_T, _B, _H), jnp.float32),
            jax.ShapeDtypeStruct((_T, _B, _H), jnp.float32),
        ],
        scratch_shapes=[pltpu.VMEM((_B, _H), jnp.float32)] * 4,
    )(gates, gates, whhf, whhb)


def _proj_body(xf_ref, xb_ref, wf_ref, wb_ref, b_ref, o_ref):
    o_ref[...] = (xf_ref[...] @ wf_ref[...] + xb_ref[...] @ wb_ref[...]
                  + b_ref[...])


def _proj(xf2d, xb2d, wf, wb, b, rows_per_step):
    n_in, n_out = wf.shape
    return pl.pallas_call(
        _proj_body,
        grid=(xf2d.shape[0] // rows_per_step,),
        in_specs=[
            pl.BlockSpec((rows_per_step, n_in), lambda i: (i, 0)),
            pl.BlockSpec((rows_per_step, n_in), lambda i: (i, 0)),
            pl.BlockSpec((n_in, n_out), lambda i: (0, 0)),
            pl.BlockSpec((n_in, n_out), lambda i: (0, 0)),
            pl.BlockSpec((1, n_out), lambda i: (0, 0)),
        ],
        out_specs=pl.BlockSpec((rows_per_step, n_out), lambda i: (i, 0)),
        out_shape=jax.ShapeDtypeStruct((xf2d.shape[0], n_out), jnp.float32),
    )(xf2d, xb2d, wf, wb, b)


def _rec1_body(gf_ref, gb_ref, whhf, whhb, w1, b1, w2, b2, out_ref,
               hf, cf, hb, cb):
    s = pl.program_id(0)

    @pl.when(s == 0)
    def _init():
        z = jnp.zeros((_B, _H), jnp.float32)
        hf[...] = z
        cf[...] = z
        hb[...] = z
        cb[...] = z

    for tt in range(_TR):
        hfv = _lstm_step(gf_ref[tt], hf, cf, whhf)
        hbv = _lstm_step(gb_ref[_TR - 1 - tt], hb, cb, whhb)

    @pl.when(s == _T // _TR - 1)
    def _cls():
        hcat = jnp.concatenate([hfv, hbv], axis=1)
        hid = jnp.maximum(hcat @ w1[...] + b1[...], 0.0)
        out_ref[...] = hid @ w2[...] + b2[...]


def _rec1(gates, whhf, whhb, w1, b1, w2, b2):
    const2 = lambda s: (0, 0)
    nsteps = _T // _TR
    return pl.pallas_call(
        _rec1_body,
        grid=(nsteps,),
        in_specs=[
            pl.BlockSpec((_TR, _B, 4 * _H), lambda s: (s, 0, 0)),
            pl.BlockSpec((_TR, _B, 4 * _H), lambda s: (nsteps - 1 - s, 0, 1)),
            pl.BlockSpec((_H, 4 * _H), const2),
            pl.BlockSpec((_H, 4 * _H), const2),
            pl.BlockSpec((2 * _H, _H), const2),
            pl.BlockSpec((1, _H), const2),
            pl.BlockSpec((_H, _NCLS), const2),
            pl.BlockSpec((1, _NCLS), const2),
        ],
        out_specs=pl.BlockSpec((_B, _NCLS), const2),
        out_shape=jax.ShapeDtypeStruct((_B, _NCLS), jnp.float32),
        scratch_shapes=[pltpu.VMEM((_B, _H), jnp.float32)] * 4,
    )(gates, gates, whhf, whhb, w1, b1, w2, b2)


def kernel(x_temporal, gcn_W0, gcn_b0, gcn_W1, gcn_b1, gcn_W2, gcn_b2,
           lstm_fw_Wih0, lstm_fw_Whh0, lstm_fw_b0,
           lstm_bw_Wih0, lstm_bw_Whh0, lstm_bw_b0,
           lstm_fw_Wih1, lstm_fw_Whh1, lstm_fw_b1,
           lstm_bw_Wih1, lstm_bw_Whh1, lstm_bw_b1,
           cls_W1, cls_b1, cls_W2, cls_b2):
    dis = np.repeat(_deg_isqrt_np()[:, None], _F, axis=1)
    disj = jnp.asarray(dis)
    disi = jnp.asarray(dis)

    wih0 = jnp.concatenate([lstm_fw_Wih0.T, lstm_bw_Wih0.T], axis=1)
    bih0 = jnp.concatenate([lstm_fw_b0, lstm_bw_b0])[None]
    g0 = _gcn(x_temporal, disj, disi, gcn_W0, gcn_b0[None], gcn_W1,
              gcn_b1[None], gcn_W2, gcn_b2[None], wih0, bih0)

    fw0, bw0 = _rec0(g0, lstm_fw_Whh0.T, lstm_bw_Whh0.T)

    wih1 = jnp.concatenate([lstm_fw_Wih1.T, lstm_bw_Wih1.T], axis=1)
    bih1 = jnp.concatenate([lstm_fw_b1, lstm_bw_b1])[None]
    gates1 = _proj(fw0.reshape(_G, _H), bw0.reshape(_G, _H),
                   wih1[:_H], wih1[_H:], bih1, 1600)
    g1 = gates1.reshape(_T, _B, 8 * _H)

    return _rec1(g1, lstm_fw_Whh1.T, lstm_bw_Whh1.T,
                 cls_W1, cls_b1[None], cls_W2, cls_b2[None])


# speedup vs baseline: 41.7448x; 1.0246x over previous
import numpy as np
import jax
import jax.numpy as jnp
from jax.experimental import pallas as pl
from jax.experimental.pallas import tpu as pltpu

_B, _T, _N, _F = 64, 50, 68, 128
_H = 256
_K = 5
_NP = _N + 2 * _K
_G = _B * _T
_GC = 64
_NCLS = 500
_ND = 2 * _K + 1


_NP2 = _N + 2 * _K
_TS = 5
_BC = 8
_TR = 10


def _deg_isqrt_np():
    deg = np.array([min(_N - 1, i + _K) - max(0, i - _K) + 1 for i in range(_N)],
                   np.float32)
    return (1.0 / np.sqrt(deg)).astype(np.float32)


def _gcn_body(x_ref, disj_ref, disi_ref, w0, b0, w1, b1, w2, b2, out_ref):
    zpad = jnp.zeros((_TS, _K, _BC, _F), jnp.float32)
    disj = disj_ref[...][None, :, None]
    disi = disi_ref[...][None, :, None]
    for tc in range(_T // _TS):
        h = jnp.transpose(x_ref[:, tc * _TS:(tc + 1) * _TS], (1, 2, 0, 3))
        for w_ref, b_ref in ((w0, b0), (w1, b1), (w2, b2)):
            hw = jnp.reshape(jnp.reshape(h, (_TS * _N * _BC, _F)) @ w_ref[...],
                             (_TS, _N, _BC, _F))
            p = jnp.concatenate([zpad, hw * disj, zpad], axis=1)
            p2 = p[:, 0:_NP2 - 1] + p[:, 1:_NP2]
            p4 = p2[:, 0:_NP2 - 3] + p2[:, 2:_NP2 - 1]
            p8 = p4[:, 0:_N] + p4[:, 4:_N + 4]
            win = p8 + p2[:, 8:_N + 8] + p[:, 10:_N + 10]
            h = jnp.maximum(win * disi + b_ref[...][None, None], 0.0)
        emb = jnp.sum(h, axis=1) * (1.0 / _N)
        out_ref[tc * _TS:(tc + 1) * _TS] = emb


def _gcn(x4d, disj, disi, w0, b0, w1, b1, w2, b2):
    const2 = lambda s: (0, 0)
    return pl.pallas_call(
        _gcn_body,
        grid=(_B // _BC,),
        in_specs=[
            pl.BlockSpec((_BC, _T, _N, _F), lambda c: (c, 0, 0, 0)),
            pl.BlockSpec((_N, _F), const2),
            pl.BlockSpec((_N, _F), const2),
            pl.BlockSpec((_F, _F), const2), pl.BlockSpec((1, _F), const2),
            pl.BlockSpec((_F, _F), const2), pl.BlockSpec((1, _F), const2),
            pl.BlockSpec((_F, _F), const2), pl.BlockSpec((1, _F), const2),
        ],
        out_specs=pl.BlockSpec((_T, _BC, _F), lambda c: (0, c, 0)),
        out_shape=jax.ShapeDtypeStruct((_T, _B, _F), jnp.float32),
    )(x4d, disj, disi, w0, b0, w1, b1, w2, b2)


def _lstm_step(x, h_ref, c_ref, w_ref, b_ref):
    g = jnp.concatenate([x, h_ref[...]], axis=1) @ w_ref[...] + b_ref[...]
    i = jax.nn.sigmoid(g[:, 0:_H])
    f = jax.nn.sigmoid(g[:, _H:2 * _H])
    gg = jnp.tanh(g[:, 2 * _H:3 * _H])
    o = jax.nn.sigmoid(g[:, 3 * _H:4 * _H])
    c = f * c_ref[...] + i * gg
    h = o * jnp.tanh(c)
    c_ref[...] = c
    h_ref[...] = h
    return h


def _rec0_body(xf_ref, xb_ref, wf, bf, wb, bb, outf_ref, outb_ref,
               hf, cf, hb, cb):
    s = pl.program_id(0)

    @pl.when(s == 0)
    def _init():
        z = jnp.zeros((_B, _H), jnp.float32)
        hf[...] = z
        cf[...] = z
        hb[...] = z
        cb[...] = z

    for tt in range(_TR):
        outf_ref[tt] = _lstm_step(xf_ref[tt], hf, cf, wf, bf)
        outb_ref[_TR - 1 - tt] = _lstm_step(xb_ref[_TR - 1 - tt], hb, cb,
                                            wb, bb)


def _rec0(emb, wf, bf, wb, bb):
    const2 = lambda s: (0, 0)
    nsteps = _T // _TR
    return pl.pallas_call(
        _rec0_body,
        grid=(nsteps,),
        in_specs=[
            pl.BlockSpec((_TR, _B, _F), lambda s: (s, 0, 0)),
            pl.BlockSpec((_TR, _B, _F), lambda s: (nsteps - 1 - s, 0, 0)),
            pl.BlockSpec((_F + _H, 4 * _H), const2),
            pl.BlockSpec((1, 4 * _H), const2),
            pl.BlockSpec((_F + _H, 4 * _H), const2),
            pl.BlockSpec((1, 4 * _H), const2),
        ],
        out_specs=[
            pl.BlockSpec((_TR, _B, _H), lambda s: (s, 0, 0)),
            pl.BlockSpec((_TR, _B, _H), lambda s: (nsteps - 1 - s, 0, 0)),
        ],
        out_shape=[
            jax.ShapeDtypeStruct((_T, _B, _H), jnp.float32),
            jax.ShapeDtypeStruct((_T, _B, _H), jnp.float32),
        ],
        scratch_shapes=[pltpu.VMEM((_B, _H), jnp.float32)] * 4,
    )(emb, emb, wf, bf, wb, bb)


def _rec1_body(fa_ref, ba_ref, fd_ref, bd_ref, wf, bf, wb, bb,
               w1, b1, w2, b2, out_ref, hf, cf, hb, cb):
    s = pl.program_id(0)

    @pl.when(s == 0)
    def _init():
        z = jnp.zeros((_B, _H), jnp.float32)
        hf[...] = z
        cf[...] = z
        hb[...] = z
        cb[...] = z

    for tt in range(_TR):
        xf = jnp.concatenate([fa_ref[tt], ba_ref[tt]], axis=1)
        hfv = _lstm_step(xf, hf, cf, wf, bf)
        xb = jnp.concatenate([fd_ref[_TR - 1 - tt], bd_ref[_TR - 1 - tt]],
                             axis=1)
        hbv = _lstm_step(xb, hb, cb, wb, bb)

    @pl.when(s == _T // _TR - 1)
    def _cls():
        hcat = jnp.concatenate([hfv, hbv], axis=1)
        hid = jnp.maximum(hcat @ w1[...] + b1[...], 0.0)
        out_ref[...] = hid @ w2[...] + b2[...]


def _rec1(fw0, bw0, wf, bf, wb, bb, w1, b1, w2, b2):
    const2 = lambda s: (0, 0)
    nsteps = _T // _TR
    asc = lambda s: (s, 0, 0)
    dsc = lambda s: (nsteps - 1 - s, 0, 0)
    return pl.pallas_call(
        _rec1_body,
        grid=(nsteps,),
        in_specs=[
            pl.BlockSpec((_TR, _B, _H), asc),
            pl.BlockSpec((_TR, _B, _H), asc),
            pl.BlockSpec((_TR, _B, _H), dsc),
            pl.BlockSpec((_TR, _B, _H), dsc),
            pl.BlockSpec((2 * _H + _H, 4 * _H), const2),
            pl.BlockSpec((1, 4 * _H), const2),
            pl.BlockSpec((2 * _H + _H, 4 * _H), const2),
            pl.BlockSpec((1, 4 * _H), const2),
            pl.BlockSpec((2 * _H, _H), const2),
            pl.BlockSpec((1, _H), const2),
            pl.BlockSpec((_H, _NCLS), const2),
            pl.BlockSpec((1, _NCLS), const2),
        ],
        out_specs=pl.BlockSpec((_B, _NCLS), const2),
        out_shape=jax.ShapeDtypeStruct((_B, _NCLS), jnp.float32),
        scratch_shapes=[pltpu.VMEM((_B, _H), jnp.float32)] * 4,
    )(fw0, bw0, fw0, bw0, wf, bf, wb, bb, w1, b1, w2, b2)


def kernel(x_temporal, gcn_W0, gcn_b0, gcn_W1, gcn_b1, gcn_W2, gcn_b2,
           lstm_fw_Wih0, lstm_fw_Whh0, lstm_fw_b0,
           lstm_bw_Wih0, lstm_bw_Whh0, lstm_bw_b0,
           lstm_fw_Wih1, lstm_fw_Whh1, lstm_fw_b1,
           lstm_bw_Wih1, lstm_bw_Whh1, lstm_bw_b1,
           cls_W1, cls_b1, cls_W2, cls_b2):
    dis = np.repeat(_deg_isqrt_np()[:, None], _F, axis=1)
    disj = jnp.asarray(dis)
    disi = jnp.asarray(dis)

    emb = _gcn(x_temporal, disj, disi, gcn_W0, gcn_b0[None], gcn_W1,
               gcn_b1[None], gcn_W2, gcn_b2[None])

    w0f = jnp.concatenate([lstm_fw_Wih0.T, lstm_fw_Whh0.T], axis=0)
    w0b = jnp.concatenate([lstm_bw_Wih0.T, lstm_bw_Whh0.T], axis=0)
    fw0, bw0 = _rec0(emb, w0f, lstm_fw_b0[None], w0b, lstm_bw_b0[None])

    w1f = jnp.concatenate([lstm_fw_Wih1.T, lstm_fw_Whh1.T], axis=0)
    w1b = jnp.concatenate([lstm_bw_Wih1.T, lstm_bw_Whh1.T], axis=0)
    return _rec1(fw0, bw0, w1f, lstm_fw_b1[None], w1b, lstm_bw_b1[None],
                 cls_W1, cls_b1[None], cls_W2, cls_b2[None])
